# baseline (speedup 1.0000x reference)
import functools

import jax
import jax.numpy as jnp
from jax import lax
from jax.experimental import pallas as pl
from jax.experimental.pallas import tpu as pltpu
from jax.experimental.pallas import tpu_sc as plsc


def _sc_embed(idx_flat, k_emb, v_emb):
    B = idx_flat.shape[0]
    k_dim = k_emb.shape[1]
    v_dim = v_emb.shape[1]
    info = plsc.get_sparse_core_info()
    nw = info.num_cores * info.num_subcores
    b_per_w = B // nw
    assert b_per_w * nw == B and (b_per_w % 8) == 0

    mesh = plsc.VectorSubcoreMesh(core_axis_name="c", subcore_axis_name="s")

    @functools.partial(
        pl.kernel,
        mesh=mesh,
        compiler_params=pltpu.CompilerParams(use_tc_tiling_on_sc=False),
        out_type=[
            jax.ShapeDtypeStruct((B, k_dim), jnp.float32),
            jax.ShapeDtypeStruct((B, v_dim), jnp.float32),
        ],
        scratch_types=[
            pltpu.VMEM((b_per_w,), jnp.int32),
            pltpu.VMEM((b_per_w, k_dim), jnp.float32),
            pltpu.VMEM((b_per_w, v_dim), jnp.float32),
            pltpu.SemaphoreType.DMA,
        ],
    )
    def body(idx_hbm, k_hbm, v_hbm, k_out, v_out, idx_v, k_rows, v_rows, sem):
        wid = lax.axis_index("s") * info.num_cores + lax.axis_index("c")
        base = wid * b_per_w
        pltpu.sync_copy(idx_hbm.at[pl.ds(base, b_per_w)], idx_v)
        ck = pltpu.async_copy(k_hbm.at[idx_v], k_rows, sem)
        cv = pltpu.async_copy(v_hbm.at[idx_v], v_rows, sem)
        ck.wait()
        cv.wait()
        pltpu.sync_copy(k_rows, k_out.at[pl.ds(base, b_per_w)])
        pltpu.sync_copy(v_rows, v_out.at[pl.ds(base, b_per_w)])

    return body(idx_flat, k_emb, v_emb)


def kernel(idx, k_emb, v_emb):
    idx_flat = idx.reshape(-1).astype(jnp.int32)
    k_out, v_out = _sc_embed(idx_flat, k_emb, v_emb)
    return (
        k_out.reshape(*idx.shape, k_emb.shape[1]),
        v_out.reshape(*idx.shape, v_emb.shape[1]),
    )


# speedup vs baseline: 1.0211x; 1.0211x over previous
import functools

import jax
import jax.numpy as jnp
from jax import lax
from jax.experimental import pallas as pl
from jax.experimental.pallas import tpu as pltpu
from jax.experimental.pallas import tpu_sc as plsc


def _sc_gather(idx_flat, table):
    B = idx_flat.shape[0]
    dim = table.shape[1]
    info = plsc.get_sparse_core_info()
    nw = info.num_cores * info.num_subcores
    b_per_w = B // nw
    assert b_per_w * nw == B and (b_per_w % 8) == 0

    mesh = plsc.VectorSubcoreMesh(core_axis_name="c", subcore_axis_name="s")

    @functools.partial(
        pl.kernel,
        mesh=mesh,
        compiler_params=pltpu.CompilerParams(use_tc_tiling_on_sc=False),
        out_type=[
            jax.ShapeDtypeStruct((B, dim), jnp.float32),
        ],
        scratch_types=[
            pltpu.VMEM((b_per_w,), jnp.int32),
            pltpu.VMEM((b_per_w, dim), jnp.float32),
            pltpu.SemaphoreType.DMA,
        ],
    )
    def body(idx_hbm, t_hbm, out_hbm, idx_v, rows, sem):
        wid = lax.axis_index("s") * info.num_cores + lax.axis_index("c")
        base = wid * b_per_w
        pltpu.sync_copy(idx_hbm.at[pl.ds(base, b_per_w)], idx_v)
        pltpu.async_copy(t_hbm.at[idx_v], rows, sem).wait()
        pltpu.sync_copy(rows, out_hbm.at[pl.ds(base, b_per_w)])

    (out,) = body(idx_flat, table)
    return out


def kernel(idx, k_emb, v_emb):
    idx_flat = idx.reshape(-1).astype(jnp.int32)
    v_out = _sc_gather(idx_flat, v_emb)
    k_out = _sc_gather(idx_flat, k_emb)
    return (
        k_out.reshape(*idx.shape, k_emb.shape[1]),
        v_out.reshape(*idx.shape, v_emb.shape[1]),
    )
